# baseline (speedup 1.0000x reference)
import jax
import jax.numpy as jnp
from jax import lax
from jax.experimental import pallas as pl
from jax.experimental.pallas import tpu as pltpu
from jax.experimental.pallas import tpu_sc as plsc

NUM_CP = 16
B, H, W, C = 4, 384, 384, 96
OUT_H, OUT_W = H // 2, W // 2
P = OUT_H * OUT_W
NPIX = B * P
NBLK = 8
BP = P // NBLK

NW = 32
PPW = NPIX // NW
CH = 96
NCHUNK = PPW // CH


def _tc_body(dpx_ref, dpy_ref, linv_t_ref, rm_ref, idx_ref, w_ref):
    coef_x = jnp.dot(dpx_ref[...], linv_t_ref[...],
                     preferred_element_type=jnp.float32)
    coef_y = jnp.dot(dpy_ref[...], linv_t_ref[...],
                     preferred_element_type=jnp.float32)
    rm = rm_ref[...]
    xs = jnp.dot(coef_x, rm, preferred_element_type=jnp.float32)
    ys = jnp.dot(coef_y, rm, preferred_element_type=jnp.float32)
    xi = 0.5 * (xs + 1.0) * float(W)
    yi = 0.5 * (ys + 1.0) * float(H)
    x0 = xi.astype(jnp.int32)
    y0 = yi.astype(jnp.int32)
    x0c = jnp.clip(x0, 0, W - 1)
    x1c = jnp.clip(x0 + 1, 0, W - 1)
    y0c = jnp.clip(y0, 0, H - 1)
    y1c = jnp.clip(y0 + 1, 0, H - 1)
    base = lax.broadcasted_iota(jnp.int32, (B, BP), 0) * (H * W)
    by0 = base + y0c * W
    by1 = base + y1c * W
    idx_ref[0] = by0 + x0c
    idx_ref[1] = by1 + x0c
    idx_ref[2] = by0 + x1c
    idx_ref[3] = by1 + x1c
    x0f = x0c.astype(jnp.float32)
    x1f = x1c.astype(jnp.float32)
    y0f = y0c.astype(jnp.float32)
    y1f = y1c.astype(jnp.float32)
    w_ref[0] = (x1f - xi) * (y1f - yi)
    w_ref[1] = (x1f - xi) * (yi - y0f)
    w_ref[2] = (xi - x0f) * (y1f - yi)
    w_ref[3] = (xi - x0f) * (yi - y0f)


_tc_call = pl.pallas_call(
    _tc_body,
    grid=(NBLK,),
    in_specs=[
        pl.BlockSpec((B, NUM_CP), lambda j: (0, 0)),
        pl.BlockSpec((B, NUM_CP), lambda j: (0, 0)),
        pl.BlockSpec((NUM_CP, NUM_CP + 3), lambda j: (0, 0)),
        pl.BlockSpec((NUM_CP + 3, BP), lambda j: (0, j)),
    ],
    out_specs=[
        pl.BlockSpec((4, B, BP), lambda j: (0, 0, j)),
        pl.BlockSpec((4, B, BP), lambda j: (0, 0, j)),
    ],
    out_shape=[
        jax.ShapeDtypeStruct((4, B, P), jnp.int32),
        jax.ShapeDtypeStruct((4, B, P), jnp.float32),
    ],
)


def _make_sc_body(npix):
  ppw = npix // NW
  nchunk = ppw // CH

  def _sc_body(img_hbm, idx_hbm, w_hbm, out_hbm,
               i0, i1, w0, w1,
               ra0, rb0, rc0, rd0, ra1, rb1, rc1, rd1,
               ov0, ov1,
               si0, si1, sw0, sw1, sr0, sr1, so0, so1):
    wid = lax.axis_index("s") * 2 + lax.axis_index("c")
    wbase = wid * ppw
    img_flat = img_hbm
    rows0 = (ra0, rb0, rc0, rd0)
    rows1 = (ra1, rb1, rc1, rd1)

    def fire_idx(chunk, ibuf, sem):
        base = wbase + chunk * CH
        for k in range(4):
            pltpu.async_copy(idx_hbm.at[pl.ds(k * npix + base, CH)],
                             ibuf.at[k], sem)

    def wait_idx(ibuf, sem):
        for k in range(4):
            pltpu.make_async_copy(idx_hbm.at[pl.ds(0, CH)], ibuf.at[k],
                                  sem).wait()

    def fire_w(chunk, wbuf, sem):
        base = wbase + chunk * CH
        for k in range(4):
            pltpu.async_copy(w_hbm.at[pl.ds(k * npix + base, CH)],
                             wbuf.at[k], sem)

    def wait_w(wbuf, sem):
        for k in range(4):
            pltpu.make_async_copy(w_hbm.at[pl.ds(0, CH)], wbuf.at[k],
                                  sem).wait()

    def fire_rows(ibuf, rows, sem):
        for k in range(4):
            pltpu.async_copy(img_flat.at[ibuf.at[k]], rows[k], sem)

    def wait_rows(rows, sem):
        for k in range(4):
            pltpu.make_async_copy(img_flat.at[pl.ds(0, CH)], rows[k], sem).wait()

    def fire_out(chunk, ov, sem):
        base = wbase + chunk * CH
        pltpu.async_copy(ov, out_hbm.at[pl.ds(base, CH)], sem)

    def wait_out(ov, sem):
        pltpu.make_async_copy(ov, out_hbm.at[pl.ds(0, CH)], sem).wait()

    def compute(wbuf, rows, ov):
        ra, rb, rc, rd = rows

        def grp(g, carry2):
            va = wbuf[0, pl.ds(g * 16, 16)]
            vb = wbuf[1, pl.ds(g * 16, 16)]
            vc = wbuf[2, pl.ds(g * 16, 16)]
            vd = wbuf[3, pl.ds(g * 16, 16)]
            for p in range(16):
                pix = g * 16 + p
                sel = jnp.full((16,), p, jnp.int32)
                sa = jnp.take_along_axis(va, sel, axis=0)
                sb = jnp.take_along_axis(vb, sel, axis=0)
                sc = jnp.take_along_axis(vc, sel, axis=0)
                sd = jnp.take_along_axis(vd, sel, axis=0)
                for c in range(0, C, 16):
                    sl = pl.ds(c, 16)
                    ov[pix, sl] = (sa * ra[pix, sl] + sb * rb[pix, sl]
                                   + sc * rc[pix, sl] + sd * rd[pix, sl])
            return carry2

        lax.fori_loop(0, CH // 16, grp, 0)

    fire_idx(0, i0, si0)
    fire_idx(1, i1, si1)
    fire_w(0, w0, sw0)
    fire_w(1, w1, sw1)
    wait_idx(i0, si0)
    fire_rows(i0, rows0, sr0)

    def stage(c, slot_cur, slot_nxt, j):
        i_c, w_c, rows_c, si_c, sw_c, sr_c, ov_c, so_c = slot_cur
        i_n, w_n, rows_n, si_n, sw_n, sr_n, ov_n, so_n = slot_nxt
        wait_idx(i_n, si_n)
        fire_rows(i_n, rows_n, sr_n)
        wait_rows(rows_c, sr_c)
        fire_idx(jnp.minimum(c + 2, nchunk - 1), i_c, si_c)
        wait_w(w_c, sw_c)

        @pl.when(j > 0)
        def _():
            wait_out(ov_c, so_c)

        compute(w_c, rows_c, ov_c)
        fire_out(c, ov_c, so_c)
        fire_w(jnp.minimum(c + 2, nchunk - 1), w_c, sw_c)

    slot0 = (i0, w0, rows0, si0, sw0, sr0, ov0, so0)
    slot1 = (i1, w1, rows1, si1, sw1, sr1, ov1, so1)

    def pair(j, carry):
        c0 = 2 * j
        stage(c0, slot0, slot1, j)
        stage(c0 + 1, slot1, slot0, j)
        return carry

    lax.fori_loop(0, nchunk // 2, pair, 0)
    wait_rows(rows0, sr0)
    wait_idx(i1, si1)
    wait_w(w0, sw0)
    wait_w(w1, sw1)
    wait_out(ov0, so0)
    wait_out(ov1, so1)

  return _sc_body


def _make_sc_call(npix):
    return pl.kernel(
        _make_sc_body(npix),
        out_type=jax.ShapeDtypeStruct((npix, C), jnp.float32),
        mesh=plsc.VectorSubcoreMesh(core_axis_name="c", subcore_axis_name="s"),
        compiler_params=pltpu.CompilerParams(
            needs_layout_passes=False, use_tc_tiling_on_sc=False),
        scratch_types=(
            [pltpu.VMEM((4, CH), jnp.int32)] * 2
            + [pltpu.VMEM((4, CH), jnp.float32)] * 2
            + [pltpu.VMEM((CH, C), jnp.float32)] * 10
            + [pltpu.SemaphoreType.DMA] * 8
        ),
    )


def kernel(image, dest_offsets, right_mat, L_inv, source_points):
    dest_points = source_points[None, :, :] + dest_offsets.reshape(B, 2, NUM_CP)
    dp_x = dest_points[:, 0, :]
    dp_y = dest_points[:, 1, :]
    linv_t = L_inv[:, 3:].T
    idx_bp, w_bp = _tc_call(dp_x, dp_y, linv_t, right_mat)
    idx4 = idx_bp.reshape(4, NPIX)
    w4 = w_bp.reshape(4, NPIX)
    hp = NPIX // 2
    outs = []
    for h in range(2):
        img_h = image[2 * h:2 * h + 2].reshape(2 * H * W, C)
        idx_h = (idx4[:, h * hp:(h + 1) * hp] - h * 2 * H * W).reshape(4 * hp)
        w_h = w4[:, h * hp:(h + 1) * hp].reshape(4 * hp)
        outs.append(_make_sc_call(hp)(img_h, idx_h, w_h))
    out_flat = jnp.concatenate(outs, axis=0)
    return out_flat.reshape(B, OUT_H, OUT_W, C)


# speedup vs baseline: 1.2719x; 1.2719x over previous
import jax
import jax.numpy as jnp
from jax import lax
from jax.experimental import pallas as pl
from jax.experimental.pallas import tpu as pltpu
from jax.experimental.pallas import tpu_sc as plsc

NUM_CP = 16
B, H, W, C = 4, 384, 384, 96
OUT_H, OUT_W = H // 2, W // 2
P = OUT_H * OUT_W
NPIX = B * P
NBLK = 8
BP = P // NBLK

NW = 32
PPW = NPIX // NW
CH = 96
NCHUNK = PPW // CH


def _tc_body(dpx_ref, dpy_ref, linv_t_ref, rm_ref, idx_ref, w_ref):
    coef_x = jnp.dot(dpx_ref[...], linv_t_ref[...],
                     preferred_element_type=jnp.float32)
    coef_y = jnp.dot(dpy_ref[...], linv_t_ref[...],
                     preferred_element_type=jnp.float32)
    rm = rm_ref[...]
    xs = jnp.dot(coef_x, rm, preferred_element_type=jnp.float32)
    ys = jnp.dot(coef_y, rm, preferred_element_type=jnp.float32)
    xi = 0.5 * (xs + 1.0) * float(W)
    yi = 0.5 * (ys + 1.0) * float(H)
    x0 = xi.astype(jnp.int32)
    y0 = yi.astype(jnp.int32)
    x0c = jnp.clip(x0, 0, W - 1)
    x1c = jnp.clip(x0 + 1, 0, W - 1)
    y0c = jnp.clip(y0, 0, H - 1)
    y1c = jnp.clip(y0 + 1, 0, H - 1)
    base = lax.broadcasted_iota(jnp.int32, (B, BP), 0) * (H * W)
    by0 = base + y0c * W
    by1 = base + y1c * W
    idx_ref[0] = by0 + x0c
    idx_ref[1] = by1 + x0c
    idx_ref[2] = by0 + x1c
    idx_ref[3] = by1 + x1c
    x0f = x0c.astype(jnp.float32)
    x1f = x1c.astype(jnp.float32)
    y0f = y0c.astype(jnp.float32)
    y1f = y1c.astype(jnp.float32)
    w_ref[0] = (x1f - xi) * (y1f - yi)
    w_ref[1] = (x1f - xi) * (yi - y0f)
    w_ref[2] = (xi - x0f) * (y1f - yi)
    w_ref[3] = (xi - x0f) * (yi - y0f)


_tc_call = pl.pallas_call(
    _tc_body,
    grid=(NBLK,),
    in_specs=[
        pl.BlockSpec((B, NUM_CP), lambda j: (0, 0)),
        pl.BlockSpec((B, NUM_CP), lambda j: (0, 0)),
        pl.BlockSpec((NUM_CP, NUM_CP + 3), lambda j: (0, 0)),
        pl.BlockSpec((NUM_CP + 3, BP), lambda j: (0, j)),
    ],
    out_specs=[
        pl.BlockSpec((4, B, BP), lambda j: (0, 0, j)),
        pl.BlockSpec((4, B, BP), lambda j: (0, 0, j)),
    ],
    out_shape=[
        jax.ShapeDtypeStruct((4, B, P), jnp.int32),
        jax.ShapeDtypeStruct((4, B, P), jnp.float32),
    ],
)


def _sc_body(img_hbm, idx_hbm, w_hbm, out_hbm,
             i0, i1, w0, w1,
             ra0, rb0, rc0, rd0, ra1, rb1, rc1, rd1,
             ov0, ov1,
             si0, si1, sw0, sw1, sr0, sr1, so0, so1):
    wid = lax.axis_index("s") * 2 + lax.axis_index("c")
    wbase = wid * PPW
    img_flat = img_hbm
    rows0 = (ra0, rb0, rc0, rd0)
    rows1 = (ra1, rb1, rc1, rd1)

    def fire_idx(chunk, ibuf, sem):
        base = wbase + chunk * CH
        for k in range(4):
            pltpu.async_copy(idx_hbm.at[pl.ds(k * NPIX + base, CH)],
                             ibuf.at[k], sem)

    def wait_idx(ibuf, sem):
        for k in range(4):
            pltpu.make_async_copy(idx_hbm.at[pl.ds(0, CH)], ibuf.at[k],
                                  sem).wait()

    def fire_w(chunk, wbuf, sem):
        base = wbase + chunk * CH
        for k in range(4):
            pltpu.async_copy(w_hbm.at[pl.ds(k * NPIX + base, CH)],
                             wbuf.at[k], sem)

    def wait_w(wbuf, sem):
        for k in range(4):
            pltpu.make_async_copy(w_hbm.at[pl.ds(0, CH)], wbuf.at[k],
                                  sem).wait()

    def fire_rows(ibuf, rows, sem):
        for k in range(4):
            pltpu.async_copy(img_flat.at[ibuf.at[k]], rows[k], sem)

    def wait_rows(rows, sem):
        for k in range(4):
            pltpu.make_async_copy(img_flat.at[pl.ds(0, CH)], rows[k], sem).wait()

    def fire_out(chunk, ov, sem):
        base = wbase + chunk * CH
        pltpu.async_copy(ov, out_hbm.at[pl.ds(base, CH)], sem)

    def wait_out(ov, sem):
        pltpu.make_async_copy(ov, out_hbm.at[pl.ds(0, CH)], sem).wait()

    def compute(wbuf, rows, ov):
        ra, rb, rc, rd = rows

        def grp(g, carry2):
            va = wbuf[0, pl.ds(g * 16, 16)]
            vb = wbuf[1, pl.ds(g * 16, 16)]
            vc = wbuf[2, pl.ds(g * 16, 16)]
            vd = wbuf[3, pl.ds(g * 16, 16)]
            for p in range(16):
                pix = g * 16 + p
                sel = jnp.full((16,), p, jnp.int32)
                sa = jnp.take_along_axis(va, sel, axis=0)
                sb = jnp.take_along_axis(vb, sel, axis=0)
                sc = jnp.take_along_axis(vc, sel, axis=0)
                sd = jnp.take_along_axis(vd, sel, axis=0)
                for c in range(0, C, 16):
                    sl = pl.ds(c, 16)
                    ov[pix, sl] = (sa * ra[pix, sl] + sb * rb[pix, sl]
                                   + sc * rc[pix, sl] + sd * rd[pix, sl])
            return carry2

        lax.fori_loop(0, CH // 16, grp, 0)

    fire_idx(0, i0, si0)
    fire_idx(1, i1, si1)
    fire_w(0, w0, sw0)
    fire_w(1, w1, sw1)
    wait_idx(i0, si0)
    fire_rows(i0, rows0, sr0)

    def stage(c, slot_cur, slot_nxt, j):
        i_c, w_c, rows_c, si_c, sw_c, sr_c, ov_c, so_c = slot_cur
        i_n, w_n, rows_n, si_n, sw_n, sr_n, ov_n, so_n = slot_nxt
        wait_idx(i_n, si_n)
        fire_rows(i_n, rows_n, sr_n)
        wait_rows(rows_c, sr_c)
        fire_idx(jnp.minimum(c + 2, NCHUNK - 1), i_c, si_c)
        wait_w(w_c, sw_c)

        @pl.when(j > 0)
        def _():
            wait_out(ov_c, so_c)

        compute(w_c, rows_c, ov_c)
        fire_out(c, ov_c, so_c)
        fire_w(jnp.minimum(c + 2, NCHUNK - 1), w_c, sw_c)

    slot0 = (i0, w0, rows0, si0, sw0, sr0, ov0, so0)
    slot1 = (i1, w1, rows1, si1, sw1, sr1, ov1, so1)

    def pair(j, carry):
        c0 = 2 * j
        stage(c0, slot0, slot1, j)
        stage(c0 + 1, slot1, slot0, j)
        return carry

    lax.fori_loop(0, NCHUNK // 2, pair, 0)
    wait_rows(rows0, sr0)
    wait_idx(i1, si1)
    wait_w(w0, sw0)
    wait_w(w1, sw1)
    wait_out(ov0, so0)
    wait_out(ov1, so1)


def _make_sc_call():
    return pl.kernel(
        _sc_body,
        out_type=jax.ShapeDtypeStruct((NPIX, C), jnp.float32),
        mesh=plsc.VectorSubcoreMesh(core_axis_name="c", subcore_axis_name="s"),
        compiler_params=pltpu.CompilerParams(
            needs_layout_passes=False, use_tc_tiling_on_sc=False),
        scratch_types=(
            [pltpu.VMEM((4, CH), jnp.int32)] * 2
            + [pltpu.VMEM((4, CH), jnp.float32)] * 2
            + [pltpu.VMEM((CH, C), jnp.float32)] * 10
            + [pltpu.SemaphoreType.DMA] * 8
        ),
    )


def kernel(image, dest_offsets, right_mat, L_inv, source_points):
    dest_points = source_points[None, :, :] + dest_offsets.reshape(B, 2, NUM_CP)
    dp_x = dest_points[:, 0, :]
    dp_y = dest_points[:, 1, :]
    linv_t = L_inv[:, 3:].T
    idx_bp, w_bp = _tc_call(dp_x, dp_y, linv_t, right_mat)
    idx4 = idx_bp.reshape(4 * NPIX)
    w4 = w_bp.reshape(4 * NPIX)
    img_flat = image.reshape(B * H * W, C)
    out_flat = _make_sc_call()(img_flat, idx4, w4)
    return out_flat.reshape(B, OUT_H, OUT_W, C)
